# speedup vs baseline: 3.1198x; 3.1198x over previous
import functools

import jax
import jax.numpy as jnp
from jax import lax
from jax.experimental import pallas as pl
from jax.experimental.pallas import tpu as pltpu
from jax.experimental.pallas import tpu_sc as plsc

_CH = 128


@functools.cache
def _make_sc_gather(V, D, B):
    info = plsc.get_sparse_core_info()
    NW = info.num_cores * info.num_subcores
    NC = info.num_cores
    assert B % (NW * _CH) == 0
    b_per_w = B // NW
    n_ch = b_per_w // _CH
    n_pairs = n_ch // 2
    assert n_ch % 2 == 0

    mesh = plsc.VectorSubcoreMesh(core_axis_name="c", subcore_axis_name="s")

    @functools.partial(
        pl.kernel,
        mesh=mesh,
        out_type=jax.ShapeDtypeStruct((B, D), jnp.float32),
        scratch_types=[
            pltpu.VMEM((n_ch, _CH), jnp.int32),
            pltpu.VMEM((2, _CH, D), jnp.float32),
            pltpu.SemaphoreType.DMA,
            pltpu.SemaphoreType.DMA,
            pltpu.SemaphoreType.DMA,
        ],
    )
    def k(idx_hbm, table_hbm, out_hbm, idx_v, rows_v, gsem, osem0, osem1):
        wid = lax.axis_index("s") * NC + lax.axis_index("c")
        base = wid * b_per_w
        pltpu.sync_copy(idx_hbm.at[wid], idx_v)
        osems = (osem0, osem1)

        def do_chunk(j, b, first):
            if not first:
                pltpu.make_async_copy(
                    rows_v.at[b],
                    out_hbm.at[pl.ds(base, _CH)],
                    osems[b],
                ).wait()
            pltpu.async_copy(
                table_hbm.at[idx_v.at[j]], rows_v.at[b], gsem
            ).wait()
            pltpu.async_copy(
                rows_v.at[b],
                out_hbm.at[pl.ds(base + j * _CH, _CH)],
                osems[b],
            )

        do_chunk(0, 0, True)
        do_chunk(1, 1, True)

        def body(p, carry):
            j = 2 * p
            do_chunk(j, 0, False)
            do_chunk(j + 1, 1, False)
            return carry

        lax.fori_loop(1, n_pairs, body, 0)

        for b in range(2):
            pltpu.make_async_copy(
                rows_v.at[b],
                out_hbm.at[pl.ds(base, _CH)],
                osems[b],
            ).wait()

    return k


def kernel(token_ids, weight):
    B0, L = token_ids.shape
    V, D = weight.shape
    B = B0 * L
    info = plsc.get_sparse_core_info()
    NW = info.num_cores * info.num_subcores
    idx = token_ids.reshape(NW, (B // NW) // _CH, _CH).astype(jnp.int32)
    out = _make_sc_gather(V, D, B)(idx, weight)
    return out.reshape(B0, L, D)


# speedup vs baseline: 3.3643x; 1.0784x over previous
import functools

import jax
import jax.numpy as jnp
from jax import lax
from jax.experimental import pallas as pl
from jax.experimental.pallas import tpu as pltpu
from jax.experimental.pallas import tpu_sc as plsc

_CH = 128
_NBUF = 5
_LAG = 3


@functools.cache
def _make_sc_gather(V, D, B):
    info = plsc.get_sparse_core_info()
    NW = info.num_cores * info.num_subcores
    NC = info.num_cores
    assert B % (NW * _CH) == 0
    b_per_w = B // NW
    n_ch = b_per_w // _CH
    assert n_ch % _NBUF == 0 and n_ch >= _NBUF
    n_grp = n_ch // _NBUF

    mesh = plsc.VectorSubcoreMesh(core_axis_name="c", subcore_axis_name="s")

    @functools.partial(
        pl.kernel,
        mesh=mesh,
        out_type=jax.ShapeDtypeStruct((B, D), jnp.float32),
        scratch_types=[
            pltpu.VMEM((n_ch, _CH), jnp.int32),
            pltpu.VMEM((_NBUF, _CH, D), jnp.float32),
            pltpu.SemaphoreType.DMA,
        ]
        + [pltpu.SemaphoreType.DMA] * _NBUF,
    )
    def k(idx_hbm, table_hbm, out_hbm, idx_v, rows_v, gsem, *osems):
        wid = lax.axis_index("s") * NC + lax.axis_index("c")
        base = wid * b_per_w
        pltpu.sync_copy(idx_hbm.at[wid], idx_v)

        def start_gather(j, b):
            pltpu.async_copy(table_hbm.at[idx_v.at[j]], rows_v.at[b], gsem)

        def wait_gather(b):
            pltpu.make_async_copy(
                table_hbm.at[idx_v.at[0]], rows_v.at[b], gsem
            ).wait()

        def start_out(j, b):
            pltpu.async_copy(
                rows_v.at[b],
                out_hbm.at[pl.ds(base + j * _CH, _CH)],
                osems[b],
            )

        def wait_out(b):
            pltpu.make_async_copy(
                rows_v.at[b], out_hbm.at[pl.ds(base, _CH)], osems[b]
            ).wait()

        for b in range(_LAG):
            start_gather(b, b)

        def body(g, carry):
            for b in range(_NBUF):
                j = g * _NBUF + b
                if b < _LAG:
                    @pl.when(g >= 1)
                    def _():
                        wait_out(b)
                        start_gather(j, b)
                else:
                    @pl.when(g >= 1)
                    def _():
                        wait_out(b)
                    start_gather(j, b)
                bl = (b - _LAG) % _NBUF
                jl = j - _LAG
                if b >= _LAG:
                    wait_gather(bl)
                    start_out(jl, bl)
                else:
                    @pl.when(g >= 1)
                    def _():
                        wait_gather(bl)
                        start_out(jl, bl)
            return carry

        lax.fori_loop(0, n_grp, body, 0, unroll=False)

        for t in range(_LAG):
            j = n_ch - _LAG + t
            b = j % _NBUF
            wait_gather(b)
            start_out(j, b)
        for b in range(_NBUF):
            wait_out(b)

    return k


def kernel(token_ids, weight):
    B0, L = token_ids.shape
    V, D = weight.shape
    B = B0 * L
    info = plsc.get_sparse_core_info()
    NW = info.num_cores * info.num_subcores
    idx = token_ids.reshape(NW, (B // NW) // _CH, _CH).astype(jnp.int32)
    out = _make_sc_gather(V, D, B)(idx, weight)
    return out.reshape(B0, L, D)


# speedup vs baseline: 5.9975x; 1.7827x over previous
import functools

import jax
import jax.numpy as jnp
from jax import lax
from jax.experimental import pallas as pl
from jax.experimental.pallas import tpu as pltpu
from jax.experimental.pallas import tpu_sc as plsc

_NBUF = 8
_LAG = 4


@functools.cache
def _make_sc_gather(V, D, B0, L):
    info = plsc.get_sparse_core_info()
    NW = info.num_cores * info.num_subcores
    NC = info.num_cores
    assert B0 % NW == 0
    n_ch = B0 // NW
    assert n_ch % _NBUF == 0 and n_ch >= _NBUF
    n_grp = n_ch // _NBUF

    mesh = plsc.VectorSubcoreMesh(core_axis_name="c", subcore_axis_name="s")

    @functools.partial(
        pl.kernel,
        mesh=mesh,
        out_type=jax.ShapeDtypeStruct((B0, L, D), jnp.float32),
        scratch_types=[
            pltpu.VMEM((n_ch, L), jnp.int32),
            pltpu.VMEM((_NBUF, L, D), jnp.float32),
            pltpu.SemaphoreType.DMA,
        ]
        + [pltpu.SemaphoreType.DMA] * _NBUF,
    )
    def k(idx_hbm, table_hbm, out_hbm, idx_v, rows_v, gsem, *osems):
        wid = lax.axis_index("s") * NC + lax.axis_index("c")
        base = wid * n_ch
        pltpu.sync_copy(idx_hbm.at[pl.ds(base, n_ch)], idx_v)

        def start_gather(j, b):
            pltpu.async_copy(table_hbm.at[idx_v.at[j]], rows_v.at[b], gsem)

        def wait_gather(b):
            pltpu.make_async_copy(
                table_hbm.at[idx_v.at[0]], rows_v.at[b], gsem
            ).wait()

        def start_out(j, b):
            pltpu.async_copy(rows_v.at[b], out_hbm.at[base + j], osems[b])

        def wait_out(b):
            pltpu.make_async_copy(
                rows_v.at[b], out_hbm.at[base], osems[b]
            ).wait()

        for b in range(_LAG):
            start_gather(b, b)

        def body(g, carry):
            for b in range(_NBUF):
                j = g * _NBUF + b
                if b < _LAG:
                    @pl.when(g >= 1)
                    def _():
                        wait_out(b)
                        start_gather(j, b)
                else:
                    @pl.when(g >= 1)
                    def _():
                        wait_out(b)
                    start_gather(j, b)
                bl = (b - _LAG) % _NBUF
                jl = j - _LAG
                if b >= _LAG:
                    wait_gather(bl)
                    start_out(jl, bl)
                else:
                    @pl.when(g >= 1)
                    def _():
                        wait_gather(bl)
                        start_out(jl, bl)
            return carry

        lax.fori_loop(0, n_grp, body, 0, unroll=False)

        for t in range(_LAG):
            j = n_ch - _LAG + t
            b = j % _NBUF
            wait_gather(b)
            start_out(j, b)
        for b in range(_NBUF):
            wait_out(b)

    return k


def kernel(token_ids, weight):
    B0, L = token_ids.shape
    V, D = weight.shape
    idx = token_ids.astype(jnp.int32)
    return _make_sc_gather(V, D, B0, L)(idx, weight)
